# Initial kernel scaffold; baseline (speedup 1.0000x reference)
import jax
import jax.numpy as jnp
from jax.experimental import pallas as pl


def kernel(user_feat, item_feat, edge_index_ui, edge_index_uu1, edge_index_uu2, edge_index_ii1, edge_index_ii2, user_idx, item_idx, neg_item_idx, W_T_ui, W_T_iu, W_A_ui, W_A_iu, ln_g, ln_b, sa_u_W1, sa_u_b1, sa_u_W2, sa_i_W1, sa_i_b1, sa_i_W2, ul1_W, ul1_b, ul2_W, ul2_b, il1_W, il1_b, il2_W, il2_b):
    raise NotImplementedError("write your pallas kernel here")


# speedup vs baseline: 4.2433x; 4.2433x over previous
import functools

import jax
import jax.numpy as jnp
from jax import lax
from jax.experimental import pallas as pl
from jax.experimental.pallas import tpu as pltpu
from jax.experimental.pallas import tpu_sc as plsc

F32 = jnp.float32
I32 = jnp.int32

_N = 10000
_D = 128
_E = 320000
_H = 128
_B = 1024

_NC = 2
_NS = 16
_NW = _NC * _NS
_C = 128
_NCH = _E // _C
_CPW = -(-_NCH // _NW)
_RC = 80
_NRC = _N // _RC
_RPW = -(-_NRC // _NW)
_RPC = -(-_NRC // _NS)

_BM = 400
_GRID = _N // _BM

_mesh = plsc.VectorSubcoreMesh(core_axis_name="c", subcore_axis_name="s")


def _wid():
    return lax.axis_index("s") * _NC + lax.axis_index("c")


def _zero2d(zb):
    def body(r, _):
        for k in range(8):
            zb[r, pl.ds(k * 16, 16)] = jnp.zeros((16,), F32)
        return 0
    lax.fori_loop(0, 128, body, 0)


def _perm(v, idx):
    dn = lax.GatherDimensionNumbers(offset_dims=(), collapsed_slice_dims=(0,),
                                    start_index_map=(0,))
    return lax.gather(v, idx[:, None], dn, slice_sizes=(1,),
                      mode=lax.GatherScatterMode.PROMISE_IN_BOUNDS)


def _lanesum(v, li):
    for t in (1, 2, 4, 8):
        v = v + _perm(v, (li + t) % 16)
    return v


def _att1_body(p_hbm, f_hbm, gi_hbm, fi_hbm, tok_hbm, s_hbm, att_hbm,
               gi_v, fi_v, p_v, f_v, s_v, z_v, att_s, sem):
    del tok_hbm
    w = _wid()
    sid = lax.axis_index("s")
    core = lax.axis_index("c")
    for k in range(_RC // 16):
        z_v[pl.ds(k * 16, 16)] = jnp.zeros((16,), F32)

    def zbody(c, _):
        idx = sid + _NS * c
        @pl.when(idx < _NRC)
        def _():
            pltpu.sync_copy(z_v, att_s.at[pl.ds(idx * _RC, _RC)])
        return 0
    lax.fori_loop(0, _RPC, zbody, 0)
    plsc.subcore_barrier()

    li = lax.iota(I32, 16)

    def cbody(c, _):
        cid = c * _NW + w
        @pl.when(cid < _NCH)
        def _():
            base = cid * _C
            pltpu.sync_copy(gi_hbm.at[pl.ds(base, _C)], gi_v)
            pltpu.sync_copy(fi_hbm.at[pl.ds(base, _C)], fi_v)
            cp1 = pltpu.async_copy(p_hbm.at[gi_v], p_v, sem)
            cp2 = pltpu.async_copy(f_hbm.at[fi_v], f_v, sem)
            cp1.wait()
            cp2.wait()

            def gbody(g, _2):
                sacc = jnp.zeros((16,), F32)
                for j in range(16):
                    e = g * 16 + j
                    acc = p_v[e, pl.ds(0, 16)] * f_v[e, pl.ds(0, 16)]
                    for k in range(1, 8):
                        acc = acc + (p_v[e, pl.ds(k * 16, 16)]
                                     * f_v[e, pl.ds(k * 16, 16)])
                    tot = _lanesum(acc, li)
                    sacc = jnp.where(li == j, tot, sacc)
                s_v[pl.ds(g * 16, 16)] = jnp.exp(sacc)
                return 0
            lax.fori_loop(0, _C // 16, gbody, 0)
            pltpu.sync_copy(s_v, s_hbm.at[pl.ds(base, _C)])
            pltpu.sync_copy(s_v, att_s.at[fi_v], add=True)
        return 0
    lax.fori_loop(0, _CPW, cbody, 0)
    plsc.subcore_barrier()

    def drain(c, _):
        idx = sid + _NS * c
        @pl.when(idx < _NRC)
        def _():
            pltpu.sync_copy(att_s.at[pl.ds(idx * _RC, _RC)],
                            s_v.at[pl.ds(0, _RC)])
            pltpu.sync_copy(s_v.at[pl.ds(0, _RC)],
                            att_hbm.at[pl.ds(core * _N + idx * _RC, _RC)])
        return 0
    lax.fori_loop(0, _RPC, drain, 0)


_sc_params = pltpu.CompilerParams(use_tc_tiling_on_sc=False)

_att1_call = pl.kernel(
    _att1_body,
    compiler_params=_sc_params,
    out_type=[jax.ShapeDtypeStruct((_E,), F32),
              jax.ShapeDtypeStruct((_NC * _N,), F32)],
    mesh=_mesh,
    scratch_types=[
        pltpu.VMEM((_C,), I32),
        pltpu.VMEM((_C,), I32),
        pltpu.VMEM((_C, _D), F32),
        pltpu.VMEM((_C, _D), F32),
        pltpu.VMEM((_C,), F32),
        pltpu.VMEM((_RC,), F32),
        pltpu.VMEM_SHARED((_N,), F32),
        pltpu.SemaphoreType.DMA,
    ],
)


def _att2_body(v_hbm, s_hbm, gi_hbm, fi_hbm, tok_hbm, acc_hbm,
               gi_v, fi_v, v_v, s_v, zb_v, acc_s, sem):
    del tok_hbm
    w = _wid()
    sid = lax.axis_index("s")
    core = lax.axis_index("c")
    _zero2d(zb_v)

    def zbody(c, _):
        idx = sid + _NS * c
        @pl.when(idx < _NRC)
        def _():
            pltpu.sync_copy(zb_v.at[pl.ds(0, _RC)],
                            acc_s.at[pl.ds(idx * _RC, _RC)])
        return 0
    lax.fori_loop(0, _RPC, zbody, 0)
    plsc.subcore_barrier()

    def cbody(c, _):
        cid = c * _NW + w
        @pl.when(cid < _NCH)
        def _():
            base = cid * _C
            pltpu.sync_copy(gi_hbm.at[pl.ds(base, _C)], gi_v)
            pltpu.sync_copy(fi_hbm.at[pl.ds(base, _C)], fi_v)
            pltpu.sync_copy(s_hbm.at[pl.ds(base, _C)], s_v)
            pltpu.async_copy(v_hbm.at[gi_v], v_v, sem).wait()

            def gbody(g, _2):
                s16 = s_v[pl.ds(g * 16, 16)]
                for j in range(16):
                    e = g * 16 + j
                    b = _perm(s16, jnp.full((16,), j, I32))
                    for k in range(8):
                        sl = pl.ds(k * 16, 16)
                        v_v[e, sl] = v_v[e, sl] * b
                return 0
            lax.fori_loop(0, _C // 16, gbody, 0)
            pltpu.sync_copy(v_v, acc_s.at[fi_v], add=True)
        return 0
    lax.fori_loop(0, _CPW, cbody, 0)
    plsc.subcore_barrier()

    def drain(c, _):
        idx = sid + _NS * c
        @pl.when(idx < _NRC)
        def _():
            r = pl.ds(idx * _RC, _RC)
            pltpu.sync_copy(acc_s.at[r], v_v.at[pl.ds(0, _RC)])
            pltpu.sync_copy(v_v.at[pl.ds(0, _RC)],
                            acc_hbm.at[pl.ds(core * _N + idx * _RC, _RC)])
        return 0
    lax.fori_loop(0, _RPC, drain, 0)


_att2_call = pl.kernel(
    _att2_body,
    compiler_params=_sc_params,
    out_type=jax.ShapeDtypeStruct((_NC * _N, _D), F32),
    mesh=_mesh,
    scratch_types=[
        pltpu.VMEM((_C,), I32),
        pltpu.VMEM((_C,), I32),
        pltpu.VMEM((_C, _D), F32),
        pltpu.VMEM((_C,), F32),
        pltpu.VMEM((128, 128), F32),
        pltpu.VMEM_SHARED((_N, _D), F32),
        pltpu.SemaphoreType.DMA,
    ],
)


def _deg_body(sidx_hbm, tok_hbm, deg_hbm, idx_v, idx2_v, ones_v, z_v, s_v, deg_s):
    del tok_hbm
    w = _wid()
    sid = lax.axis_index("s")
    core = lax.axis_index("c")
    for k in range(8):
        ones_v[pl.ds(k * 16, 16)] = jnp.ones((16,), F32)
    for k in range(_RC // 16):
        z_v[pl.ds(k * 16, 16)] = jnp.zeros((16,), F32)

    nzc = 8 * _N // _RC

    def zbody(c, _):
        idx = sid + _NS * c
        @pl.when(idx < nzc)
        def _():
            pltpu.sync_copy(z_v, deg_s.at[pl.ds(idx * _RC, _RC)])
        return 0
    lax.fori_loop(0, -(-nzc // _NS), zbody, 0)
    plsc.subcore_barrier()

    for g in range(8):
        def cbody(c, _):
            cid = c * _NW + w
            @pl.when(cid < _NCH)
            def _():
                pltpu.sync_copy(sidx_hbm.at[pl.ds(g * _E + cid * _C, _C)],
                                idx_v)
                for k in range(_C // 16):
                    sl = pl.ds(k * 16, 16)
                    idx2_v[sl] = idx_v[sl] + (g * _N)
                pltpu.sync_copy(ones_v, deg_s.at[idx2_v], add=True)
            return 0
        lax.fori_loop(0, _CPW, cbody, 0)
    plsc.subcore_barrier()

    def drain(c, _):
        lin = sid + _NS * c
        @pl.when(lin < nzc)
        def _():
            pltpu.sync_copy(deg_s.at[pl.ds(lin * _RC, _RC)], s_v)
            pltpu.sync_copy(s_v,
                            deg_hbm.at[pl.ds(core * 8 * _N + lin * _RC, _RC)])
        return 0
    lax.fori_loop(0, -(-nzc // _NS), drain, 0)


_deg_call = pl.kernel(
    _deg_body,
    compiler_params=_sc_params,
    out_type=jax.ShapeDtypeStruct((_NC * 8 * _N,), F32),
    mesh=_mesh,
    scratch_types=[
        pltpu.VMEM((_C,), I32),
        pltpu.VMEM((_C,), I32),
        pltpu.VMEM((_C,), F32),
        pltpu.VMEM((_RC,), F32),
        pltpu.VMEM((_RC,), F32),
        pltpu.VMEM_SHARED((8 * _N,), F32),
    ],
)


def _agg_body(m_hbm, si_hbm, di_hbm, tok_hbm, acc_hbm, si_v, di_v, rows_v, zb_v,
              acc_s, sem):
    del tok_hbm
    w = _wid()
    sid = lax.axis_index("s")
    core = lax.axis_index("c")
    _zero2d(zb_v)

    def zbody(c, _):
        idx = sid + _NS * c
        @pl.when(idx < _NRC)
        def _():
            pltpu.sync_copy(zb_v.at[pl.ds(0, _RC)],
                            acc_s.at[pl.ds(idx * _RC, _RC)])
        return 0
    lax.fori_loop(0, _RPC, zbody, 0)
    plsc.subcore_barrier()

    def cbody(c, _):
        cid = c * _NW + w
        @pl.when(cid < _NCH)
        def _():
            base = cid * _C
            pltpu.sync_copy(si_hbm.at[pl.ds(base, _C)], si_v)
            pltpu.sync_copy(di_hbm.at[pl.ds(base, _C)], di_v)
            pltpu.async_copy(m_hbm.at[si_v], rows_v, sem).wait()
            pltpu.sync_copy(rows_v, acc_s.at[di_v], add=True)
        return 0
    lax.fori_loop(0, _CPW, cbody, 0)
    plsc.subcore_barrier()

    def drain(c, _):
        idx = sid + _NS * c
        @pl.when(idx < _NRC)
        def _():
            r = pl.ds(idx * _RC, _RC)
            pltpu.sync_copy(acc_s.at[r], rows_v.at[pl.ds(0, _RC)])
            pltpu.sync_copy(rows_v.at[pl.ds(0, _RC)],
                            acc_hbm.at[pl.ds(core * _N + idx * _RC, _RC)])
        return 0
    lax.fori_loop(0, _RPC, drain, 0)


_agg_call = pl.kernel(
    _agg_body,
    compiler_params=_sc_params,
    out_type=jax.ShapeDtypeStruct((_NC * _N, _D), F32),
    mesh=_mesh,
    scratch_types=[
        pltpu.VMEM((_C,), I32),
        pltpu.VMEM((_C,), I32),
        pltpu.VMEM((_C, _D), F32),
        pltpu.VMEM((128, 128), F32),
        pltpu.VMEM_SHARED((_N, _D), F32),
        pltpu.SemaphoreType.DMA,
    ],
)


def _gat_body(ut_hbm, it_hbm, ui_hbm, ii_hbm, ni_hbm, ou, oi, on,
              idx_v, rows_v, sem):
    w = _wid()
    bpw = _B // _NW
    base = w * bpw
    for ih, tab, oh in ((ui_hbm, ut_hbm, ou),
                        (ii_hbm, it_hbm, oi),
                        (ni_hbm, it_hbm, on)):
        pltpu.sync_copy(ih.at[pl.ds(base, bpw)], idx_v)
        pltpu.async_copy(tab.at[idx_v], rows_v, sem).wait()
        pltpu.sync_copy(rows_v, oh.at[pl.ds(base, bpw)])


_gat_call = pl.kernel(
    _gat_body,
    compiler_params=_sc_params,
    out_type=[jax.ShapeDtypeStruct((_B, _D), F32)] * 3,
    mesh=_mesh,
    scratch_types=[
        pltpu.VMEM((_B // _NW,), I32),
        pltpu.VMEM((_B // _NW, _D), F32),
        pltpu.SemaphoreType.DMA,
    ],
)


def _prep_body(uf, itf, wtu, wti, wau, wai, pu, pi):
    thu = jnp.dot(uf[...], wtu[...], preferred_element_type=F32)
    pu[...] = thu * wau[...]
    thi = jnp.dot(itf[...], wti[...], preferred_element_type=F32)
    pi[...] = thi * wai[...]


_prep_call = pl.pallas_call(
    _prep_body,
    grid=(_GRID,),
    in_specs=[
        pl.BlockSpec((_BM, _D), lambda i: (i, 0)),
        pl.BlockSpec((_BM, _D), lambda i: (i, 0)),
        pl.BlockSpec((_D, _D), lambda i: (0, 0)),
        pl.BlockSpec((_D, _D), lambda i: (0, 0)),
        pl.BlockSpec((1, _D), lambda i: (0, 0)),
        pl.BlockSpec((1, _D), lambda i: (0, 0)),
    ],
    out_specs=[pl.BlockSpec((_BM, _D), lambda i: (i, 0)),
               pl.BlockSpec((_BM, _D), lambda i: (i, 0))],
    out_shape=[jax.ShapeDtypeStruct((_N, _D), F32),
               jax.ShapeDtypeStruct((_N, _D), F32)],
)


def _ln(x, g, b):
    mu = jnp.mean(x, axis=-1, keepdims=True)
    var = jnp.mean((x - mu) ** 2, axis=-1, keepdims=True)
    return (x - mu) / jnp.sqrt(var + 1e-5) * g + b


def _mid_body(au0, au1, atu0, atu1, ai0, ai1, ati0, ati1, uf, itf, lng, lnb,
              d10, d11, d20, d21, d30, d31, d40, d41,
              h1u, h1i, m1, m2, m3, m4):
    attu = atu0[...] + atu1[...]
    hu = (au0[...] + au1[...]) * jnp.where(attu > 0, 1.0 / attu, 0.0)
    h1u[...] = _ln(jax.nn.relu(hu), lng[...], lnb[...])
    atti = ati0[...] + ati1[...]
    hi = (ai0[...] + ai1[...]) * jnp.where(atti > 0, 1.0 / atti, 0.0)
    h1i[...] = _ln(jax.nn.relu(hi), lng[...], lnb[...])
    m1[...] = uf[...] * lax.rsqrt(jnp.maximum(d10[...] + d11[...], 1.0))
    m2[...] = uf[...] * lax.rsqrt(jnp.maximum(d20[...] + d21[...], 1.0))
    m3[...] = itf[...] * lax.rsqrt(jnp.maximum(d30[...] + d31[...], 1.0))
    m4[...] = itf[...] * lax.rsqrt(jnp.maximum(d40[...] + d41[...], 1.0))


_col_spec = pl.BlockSpec((_BM, 1), lambda i: (i, 0))
_row_spec = pl.BlockSpec((_BM, _D), lambda i: (i, 0))
_wide_spec = pl.BlockSpec((1, _D), lambda i: (0, 0))

_mid_call = pl.pallas_call(
    _mid_body,
    grid=(_GRID,),
    in_specs=[_row_spec, _row_spec, _col_spec, _col_spec,
              _row_spec, _row_spec, _col_spec, _col_spec,
              _row_spec, _row_spec, _wide_spec, _wide_spec,
              _col_spec, _col_spec, _col_spec, _col_spec,
              _col_spec, _col_spec, _col_spec, _col_spec],
    out_specs=[_row_spec] * 6,
    out_shape=[jax.ShapeDtypeStruct((_N, _D), F32)] * 6,
)


def _post_body(g10, g11, g20, g21, g30, g31, g40, g41,
               d10, d11, d20, d21, d30, d31, d40, d41,
               uW1, ub1, uW2, iW1, ib1, iW2,
               zu1, zu2, zi1, zi2, su1, su2, si1, si2):
    def z(a0, a1, e0, e1):
        return (a0[...] + a1[...]) * lax.rsqrt(
            jnp.maximum(e0[...] + e1[...], 1.0))

    def score(zz, W1, b1, W2):
        t = jnp.tanh(jnp.dot(zz, W1[...], preferred_element_type=F32) + b1[...])
        return jnp.full((8, _D), jnp.sum(t * W2[...]), F32)

    z1 = z(g10, g11, d10, d11)
    zu1[...] = z1
    su1[...] = score(z1, uW1, ub1, uW2)
    z2 = z(g20, g21, d20, d21)
    zu2[...] = z2
    su2[...] = score(z2, uW1, ub1, uW2)
    z3 = z(g30, g31, d30, d31)
    zi1[...] = z3
    si1[...] = score(z3, iW1, ib1, iW2)
    z4 = z(g40, g41, d40, d41)
    zi2[...] = z4
    si2[...] = score(z4, iW1, ib1, iW2)


_one_spec = pl.BlockSpec((8, _D), lambda i: (i, 0))
_w_spec = pl.BlockSpec((_D, _H), lambda i: (0, 0))

_post_call = pl.pallas_call(
    _post_body,
    grid=(_GRID,),
    in_specs=[_row_spec] * 8 + [_col_spec] * 8 +
             [_w_spec, _wide_spec, _wide_spec, _w_spec, _wide_spec, _wide_spec],
    out_specs=[_row_spec] * 4 + [_one_spec] * 4,
    out_shape=[jax.ShapeDtypeStruct((_N, _D), F32)] * 4 +
              [jax.ShapeDtypeStruct((_GRID * 8, _D), F32)] * 4,
)


def _fuse_body(zu1, zu2, zi1, zi2, h1u, h1i, uf, itf,
               su1, su2, si1, si2,
               u1W, u1b, u2W, u2b, i1W, i1b, i2W, i2b,
               ut, it):
    def beta(s1, s2):
        m1 = jnp.sum(s1[...][:, 0:1]) / (8.0 * _N)
        m2 = jnp.sum(s2[...][:, 0:1]) / (8.0 * _N)
        mx = jnp.maximum(m1, m2)
        e1 = jnp.exp(m1 - mx)
        e2 = jnp.exp(m2 - mx)
        return e1 / (e1 + e2), e2 / (e1 + e2)

    bu1, bu2 = beta(su1, su2)
    bi1, bi2 = beta(si1, si2)
    h2u = zu1[...] * bu1 + zu2[...] * bu2
    h2i = zi1[...] * bi1 + zi2[...] * bi2

    u1 =(jnp.dot(h1u[...], u1W[:_D], preferred_element_type=F32)
          + jnp.dot(h2u, u1W[_D:], preferred_element_type=F32) + u1b[...])
    ut[...] = (jnp.dot(u1, u2W[:_D], preferred_element_type=F32)
               + jnp.dot(uf[...], u2W[_D:], preferred_element_type=F32)
               + u2b[...])
    i1 = (jnp.dot(h1i[...], i1W[:_D], preferred_element_type=F32)
          + jnp.dot(h2i, i1W[_D:], preferred_element_type=F32) + i1b[...])
    it[...] = (jnp.dot(i1, i2W[:_D], preferred_element_type=F32)
               + jnp.dot(itf[...], i2W[_D:], preferred_element_type=F32)
               + i2b[...])


_sc_spec = pl.BlockSpec((_GRID * 8, _D), lambda i: (0, 0))
_w2_spec = pl.BlockSpec((2 * _D, _D), lambda i: (0, 0))

_fuse_call = pl.pallas_call(
    _fuse_body,
    grid=(_GRID,),
    in_specs=[_row_spec] * 8 + [_sc_spec] * 4 +
             [_w2_spec, _wide_spec, _w2_spec, _wide_spec,
              _w2_spec, _wide_spec, _w2_spec, _wide_spec],
    out_specs=[_row_spec] * 2,
    out_shape=[jax.ShapeDtypeStruct((_N, _D), F32)] * 2,
)


def kernel(user_feat, item_feat, edge_index_ui, edge_index_uu1, edge_index_uu2,
           edge_index_ii1, edge_index_ii2, user_idx, item_idx, neg_item_idx,
           W_T_ui, W_T_iu, W_A_ui, W_A_iu, ln_g, ln_b,
           sa_u_W1, sa_u_b1, sa_u_W2, sa_i_W1, sa_i_b1, sa_i_W2,
           ul1_W, ul1_b, ul2_W, ul2_b, il1_W, il1_b, il2_W, il2_b):
    i32 = lambda x: x.astype(I32)
    src = i32(edge_index_ui[0])
    dst = i32(edge_index_ui[1])

    pu, pi = _prep_call(user_feat, item_feat, W_T_ui, W_T_iu,
                        W_A_ui.reshape(1, _D), W_A_iu.reshape(1, _D))

    sUI, attI = _att1_call(pu, item_feat, src, dst, jnp.zeros((8,), F32))
    accI = _att2_call(user_feat, sUI, src, dst, attI[:8])
    sIU, attU = _att1_call(pi, user_feat, dst, src, accI[0, :8])
    accU = _att2_call(item_feat, sIU, dst, src, attU[:8])
    accI = accI.reshape(_NC, _N, _D)
    accU = accU.reshape(_NC, _N, _D)
    attI = attI.reshape(_NC, _N)
    attU = attU.reshape(_NC, _N)

    sidx = jnp.stack([
        i32(edge_index_uu1[0]), i32(edge_index_uu1[1]),
        i32(edge_index_uu2[0]), i32(edge_index_uu2[1]),
        i32(edge_index_ii1[0]), i32(edge_index_ii1[1]),
        i32(edge_index_ii2[0]), i32(edge_index_ii2[1]),
    ]).reshape(-1)
    degf = _deg_call(sidx, accU[0, :8])
    degr = degf.reshape(_NC, 8, _N)
    col = lambda a: a.reshape(_N, 1)

    h1u, h1i, m1, m2, m3, m4 = _mid_call(
        accU[0], accU[1], col(attU[0]), col(attU[1]),
        accI[0], accI[1], col(attI[0]), col(attI[1]),
        user_feat, item_feat, ln_g.reshape(1, _D), ln_b.reshape(1, _D),
        col(degr[0, 0]), col(degr[1, 0]), col(degr[0, 2]), col(degr[1, 2]),
        col(degr[0, 4]), col(degr[1, 4]), col(degr[0, 6]), col(degr[1, 6]))

    agg1 = _agg_call(m1, i32(edge_index_uu1[0]), i32(edge_index_uu1[1]),
                     degf[:8])
    agg2 = _agg_call(m2, i32(edge_index_uu2[0]), i32(edge_index_uu2[1]),
                     agg1[0, :8])
    agg3 = _agg_call(m3, i32(edge_index_ii1[0]), i32(edge_index_ii1[1]),
                     agg2[0, :8])
    agg4 = _agg_call(m4, i32(edge_index_ii2[0]), i32(edge_index_ii2[1]),
                     agg3[0, :8])
    agg1 = agg1.reshape(_NC, _N, _D)
    agg2 = agg2.reshape(_NC, _N, _D)
    agg3 = agg3.reshape(_NC, _N, _D)
    agg4 = agg4.reshape(_NC, _N, _D)

    zu1, zu2, zi1, zi2, su1, su2, si1, si2 = _post_call(
        agg1[0], agg1[1], agg2[0], agg2[1],
        agg3[0], agg3[1], agg4[0], agg4[1],
        col(degr[0, 1]), col(degr[1, 1]), col(degr[0, 3]), col(degr[1, 3]),
        col(degr[0, 5]), col(degr[1, 5]), col(degr[0, 7]), col(degr[1, 7]),
        sa_u_W1, sa_u_b1.reshape(1, _H), sa_u_W2.reshape(1, _H),
        sa_i_W1, sa_i_b1.reshape(1, _H), sa_i_W2.reshape(1, _H))

    ut, itb = _fuse_call(
        zu1, zu2, zi1, zi2, h1u, h1i, user_feat, item_feat,
        su1, su2, si1, si2,
        ul1_W, ul1_b.reshape(1, _D), ul2_W, ul2_b.reshape(1, _D),
        il1_W, il1_b.reshape(1, _D), il2_W, il2_b.reshape(1, _D))

    ou, oi, on = _gat_call(ut, itb, i32(user_idx), i32(item_idx),
                           i32(neg_item_idx))
    return (ou, oi, on)
